# baseline (speedup 1.0000x reference)
import jax
import jax.numpy as jnp
from jax import lax
from jax.experimental import pallas as pl
from jax.experimental.pallas import tpu as pltpu
from jax.experimental.pallas import tpu_sc as plsc

_EMB = 128
_D0, _D1, _D2 = 5, 6, 2
_NT = _D0 * _D1 * _D2
_E = 320000
_NW = 32
_PER_W = _E // _NW
_CH = 80
_NCH = _PER_W // _CH
_NB = 5
_K = 3


def _sc_body(a_hbm, b_hbm, c_hbm, w0_hbm, w1_hbm, w2_hbm, out_hbm,
             a_v, b_v, c_v, w0_v, w1_v, w2_v, t_v, t_sh, idx_v, rows_v,
             asem, gsem, osem):
    sid = lax.axis_index("s")
    wid = sid * 2 + lax.axis_index("c")
    base = wid * _PER_W

    pltpu.make_async_copy(a_hbm.at[pl.ds(base, _PER_W)], a_v, asem).start()
    pltpu.make_async_copy(b_hbm.at[pl.ds(base, _PER_W)], b_v, asem).start()
    pltpu.make_async_copy(c_hbm.at[pl.ds(base, _PER_W)], c_v, asem).start()

    pltpu.sync_copy(w0_hbm, w0_v)
    pltpu.sync_copy(w1_hbm, w1_v)
    pltpu.sync_copy(w2_hbm, w2_v)

    for k in range(_NT):
        a, b, c = k // (_D1 * _D2), (k // _D2) % _D1, k % _D2
        for j in range(_EMB // 16):
            s = pl.ds(j * 16, 16)
            t_v[k, s] = w0_v[a, s] + w1_v[b, s] + w2_v[c, s]

    @pl.when(sid == 0)
    def _stage_table():
        pltpu.sync_copy(t_v, t_sh)

    pltpu.make_async_copy(a_hbm.at[pl.ds(base, _PER_W)], a_v, asem).wait()
    pltpu.make_async_copy(b_hbm.at[pl.ds(base, _PER_W)], b_v, asem).wait()
    pltpu.make_async_copy(c_hbm.at[pl.ds(base, _PER_W)], c_v, asem).wait()

    def compute_idx(r, slot):
        for j in range(_CH // 16):
            o = r * _CH + j * 16
            av = a_v[pl.ds(o, 16)]
            bv = b_v[pl.ds(o, 16)]
            cv = c_v[pl.ds(o, 16)]
            idx_v[slot, pl.ds(j * 16, 16)] = (av * (_D1 * _D2) + bv * _D2) + cv

    def g_fire(r, slot):
        pltpu.make_async_copy(
            t_sh.at[idx_v.at[slot]], rows_v.at[slot], gsem.at[slot]).start()

    def g_wait(r, slot):
        pltpu.make_async_copy(
            t_sh.at[idx_v.at[slot]], rows_v.at[slot], gsem.at[slot]).wait()

    def c_fire(r, slot):
        pltpu.make_async_copy(
            rows_v.at[slot], out_hbm.at[pl.ds(base + r * _CH, _CH)],
            osem.at[slot]).start()

    def c_wait(r, slot):
        pltpu.make_async_copy(
            rows_v.at[slot], out_hbm.at[pl.ds(base + r * _CH, _CH)],
            osem.at[slot]).wait()

    for r0 in range(_K):
        compute_idx(r0, r0)

    plsc.subcore_barrier()

    for r0 in range(_K):
        g_fire(r0, r0)

    g_wait(0, 0); c_fire(0, 0); compute_idx(3, 3); g_fire(3, 3)
    g_wait(1, 1); c_fire(1, 1); compute_idx(4, 4); g_fire(4, 4)
    g_wait(2, 2); c_fire(2, 2); compute_idx(5, 0); c_wait(0, 0); g_fire(5, 0)
    g_wait(3, 3); c_fire(3, 3); compute_idx(6, 1); c_wait(1, 1); g_fire(6, 1)
    g_wait(4, 4); c_fire(4, 4); compute_idx(7, 2); c_wait(2, 2); g_fire(7, 2)

    def outer(g, carry):
        r0 = g * _NB
        for b in range(_NB):
            r = r0 + b
            s = (b + _K) % _NB
            g_wait(r, b)
            c_fire(r, b)
            compute_idx(r + _K, s)
            c_wait(r - (_NB - _K), s)
            g_fire(r + _K, s)
        return carry

    lax.fori_loop(1, _NCH // _NB - 1, outer, 0)

    r0 = _NCH - _NB
    g_wait(r0 + 0, 0); c_fire(r0 + 0, 0); compute_idx(r0 + 3, 3)
    c_wait(r0 - 2, 3); g_fire(r0 + 3, 3)
    g_wait(r0 + 1, 1); c_fire(r0 + 1, 1); compute_idx(r0 + 4, 4)
    c_wait(r0 - 1, 4); g_fire(r0 + 4, 4)
    g_wait(r0 + 2, 2); c_fire(r0 + 2, 2); c_wait(r0 + 0, 0)
    g_wait(r0 + 3, 3); c_fire(r0 + 3, 3); c_wait(r0 + 1, 1)
    g_wait(r0 + 4, 4); c_fire(r0 + 4, 4); c_wait(r0 + 2, 2)
    c_wait(r0 + 3, 3)
    c_wait(r0 + 4, 4)


def _sc_encode(a, b, c, W0, W1, W2):
    mesh = plsc.VectorSubcoreMesh(core_axis_name="c", subcore_axis_name="s")
    return pl.kernel(
        _sc_body,
        out_type=jax.ShapeDtypeStruct((_E, _EMB), jnp.float32),
        mesh=mesh,
        scratch_types=[
            pltpu.VMEM((_PER_W,), jnp.int32),
            pltpu.VMEM((_PER_W,), jnp.int32),
            pltpu.VMEM((_PER_W,), jnp.int32),
            pltpu.VMEM((_D0, _EMB), jnp.float32),
            pltpu.VMEM((_D1, _EMB), jnp.float32),
            pltpu.VMEM((_D2, _EMB), jnp.float32),
            pltpu.VMEM((_NT, _EMB), jnp.float32),
            pltpu.VMEM_SHARED((_NT, _EMB), jnp.float32),
            pltpu.VMEM((_NB, _CH), jnp.int32),
            pltpu.VMEM((_NB, _CH, _EMB), jnp.float32),
            pltpu.SemaphoreType.DMA,
            pltpu.SemaphoreType.DMA((_NB,)),
            pltpu.SemaphoreType.DMA((_NB,)),
        ],
    )(a, b, c, W0, W1, W2)


def kernel(edge_attr, W0, W1, W2):
    attr = edge_attr.astype(jnp.int32)
    return _sc_encode(attr[:, 0], attr[:, 1], attr[:, 2], W0, W1, W2)


# speedup vs baseline: 1.0597x; 1.0597x over previous
import jax
import jax.numpy as jnp
from jax import lax
from jax.experimental import pallas as pl
from jax.experimental.pallas import tpu as pltpu
from jax.experimental.pallas import tpu_sc as plsc

_EMB = 128
_D0, _D1, _D2 = 5, 6, 2
_NT = _D0 * _D1 * _D2
_E = 320000
_NW = 32
_PER_W = _E // _NW
_CH = 80
_NCH = _PER_W // _CH
_NB = 5
_K = 3


def _sc_body(t_hbm, a_hbm, b_hbm, c_hbm, out_hbm,
             a_v, b_v, c_v, t_v, t_sh, idx_v, rows_v,
             asem, gsem, osem):
    sid = lax.axis_index("s")
    wid = sid * 2 + lax.axis_index("c")
    base = wid * _PER_W

    pltpu.make_async_copy(a_hbm.at[pl.ds(base, _PER_W)], a_v, asem).start()
    pltpu.make_async_copy(b_hbm.at[pl.ds(base, _PER_W)], b_v, asem).start()
    pltpu.make_async_copy(c_hbm.at[pl.ds(base, _PER_W)], c_v, asem).start()

    @pl.when(sid == 0)
    def _stage_table():
        pltpu.sync_copy(t_hbm, t_v)
        pltpu.sync_copy(t_v, t_sh)

    pltpu.make_async_copy(a_hbm.at[pl.ds(base, _PER_W)], a_v, asem).wait()
    pltpu.make_async_copy(b_hbm.at[pl.ds(base, _PER_W)], b_v, asem).wait()
    pltpu.make_async_copy(c_hbm.at[pl.ds(base, _PER_W)], c_v, asem).wait()

    def compute_idx(r, slot):
        for j in range(_CH // 16):
            o = r * _CH + j * 16
            av = a_v[pl.ds(o, 16)]
            bv = b_v[pl.ds(o, 16)]
            cv = c_v[pl.ds(o, 16)]
            idx_v[slot, pl.ds(j * 16, 16)] = (av * (_D1 * _D2) + bv * _D2) + cv

    def g_fire(r, slot):
        pltpu.make_async_copy(
            t_sh.at[idx_v.at[slot]], rows_v.at[slot], gsem.at[slot]).start()

    def g_wait(r, slot):
        pltpu.make_async_copy(
            t_sh.at[idx_v.at[slot]], rows_v.at[slot], gsem.at[slot]).wait()

    def c_fire(r, slot):
        pltpu.make_async_copy(
            rows_v.at[slot], out_hbm.at[pl.ds(base + r * _CH, _CH)],
            osem.at[slot]).start()

    def c_wait(r, slot):
        pltpu.make_async_copy(
            rows_v.at[slot], out_hbm.at[pl.ds(base + r * _CH, _CH)],
            osem.at[slot]).wait()

    for r0 in range(_K):
        compute_idx(r0, r0)

    plsc.subcore_barrier()

    for r0 in range(_K):
        g_fire(r0, r0)

    g_wait(0, 0); c_fire(0, 0); compute_idx(3, 3); g_fire(3, 3)
    g_wait(1, 1); c_fire(1, 1); compute_idx(4, 4); g_fire(4, 4)
    g_wait(2, 2); c_fire(2, 2); compute_idx(5, 0); c_wait(0, 0); g_fire(5, 0)
    g_wait(3, 3); c_fire(3, 3); compute_idx(6, 1); c_wait(1, 1); g_fire(6, 1)
    g_wait(4, 4); c_fire(4, 4); compute_idx(7, 2); c_wait(2, 2); g_fire(7, 2)

    def outer(g, carry):
        r0 = g * _NB
        for b in range(_NB):
            r = r0 + b
            s = (b + _K) % _NB
            g_wait(r, b)
            c_fire(r, b)
            compute_idx(r + _K, s)
            c_wait(r - (_NB - _K), s)
            g_fire(r + _K, s)
        return carry

    lax.fori_loop(1, _NCH // _NB - 1, outer, 0)

    r0 = _NCH - _NB
    g_wait(r0 + 0, 0); c_fire(r0 + 0, 0); compute_idx(r0 + 3, 3)
    c_wait(r0 - 2, 3); g_fire(r0 + 3, 3)
    g_wait(r0 + 1, 1); c_fire(r0 + 1, 1); compute_idx(r0 + 4, 4)
    c_wait(r0 - 1, 4); g_fire(r0 + 4, 4)
    g_wait(r0 + 2, 2); c_fire(r0 + 2, 2); c_wait(r0 + 0, 0)
    g_wait(r0 + 3, 3); c_fire(r0 + 3, 3); c_wait(r0 + 1, 1)
    g_wait(r0 + 4, 4); c_fire(r0 + 4, 4); c_wait(r0 + 2, 2)
    c_wait(r0 + 3, 3)
    c_wait(r0 + 4, 4)


def _table_body(w0, w1, w2, t):
    for a in range(_D0):
        for b in range(_D1):
            for c in range(_D2):
                k = (a * _D1 + b) * _D2 + c
                t[k:k + 1, :] = w0[a:a + 1, :] + w1[b:b + 1, :] + w2[c:c + 1, :]


def _build_table(W0, W1, W2):
    return pl.pallas_call(
        _table_body,
        out_shape=jax.ShapeDtypeStruct((_NT, _EMB), jnp.float32),
    )(W0, W1, W2)


def _sc_encode(T, a, b, c):
    mesh = plsc.VectorSubcoreMesh(core_axis_name="c", subcore_axis_name="s")
    return pl.kernel(
        _sc_body,
        out_type=jax.ShapeDtypeStruct((_E, _EMB), jnp.float32),
        mesh=mesh,
        scratch_types=[
            pltpu.VMEM((_PER_W,), jnp.int32),
            pltpu.VMEM((_PER_W,), jnp.int32),
            pltpu.VMEM((_PER_W,), jnp.int32),
            pltpu.VMEM((_NT, _EMB), jnp.float32),
            pltpu.VMEM_SHARED((_NT, _EMB), jnp.float32),
            pltpu.VMEM((_NB, _CH), jnp.int32),
            pltpu.VMEM((_NB, _CH, _EMB), jnp.float32),
            pltpu.SemaphoreType.DMA,
            pltpu.SemaphoreType.DMA((_NB,)),
            pltpu.SemaphoreType.DMA((_NB,)),
        ],
    )(T, a, b, c)


def kernel(edge_attr, W0, W1, W2):
    attr = edge_attr.astype(jnp.int32)
    T = _build_table(W0, W1, W2)
    return _sc_encode(T, attr[:, 0], attr[:, 1], attr[:, 2])
